# baseline (speedup 1.0000x reference)
import functools

import jax
import jax.numpy as jnp
from jax import lax
from jax.experimental import pallas as pl
from jax.experimental.pallas import tpu as pltpu
from jax.experimental.pallas import tpu_sc as plsc

POOLED = 7
STRIDES = (4, 8, 16, 32, 64)
LEVEL_HW = (96, 48, 24, 12, 6)
CH = 192
NROI_L = 64
NROI = NROI_L * 5
BINS_PER_ROI = POOLED * POOLED
IDX_PER_ROI = BINS_PER_ROI * 4
NBINS = NROI * BINS_PER_ROI
NTILES = 32
BPT = 512
NBINS_PAD = BPT * NTILES
CB = 16
ROWS_PER_CHUNK = CB * 4
NCHUNKS = BPT // CB
NBUF = 4
NVREG = CH // 16

_BASES = []
_off = 0
for _hw in LEVEL_HW:
    _BASES.append(_off)
    _off += 2 * _hw * _hw
NROWS = _off
ZROW = NROWS
RB = 1536
NT_PAD = RB * 16
NSPAN = 4


def _idx_body(rois_ref, par_ref, idx_ref):
    col = lax.broadcasted_iota(jnp.int32, (NROI, IDX_PER_ROI), 1)
    ph = (col // 28).astype(jnp.float32)
    pw = ((col % 28) // 4).astype(jnp.float32)
    di = col % 4

    r = rois_ref[...]
    p = par_ref[...]
    b = r[:, 0:1].astype(jnp.int32)
    s = p[:, 0:1]
    hwf = p[:, 1:2]
    basef = p[:, 2:3]
    x1 = jnp.round(r[:, 1:2] * s)
    y1 = jnp.round(r[:, 2:3] * s)
    x2 = jnp.round(r[:, 3:4] * s)
    y2 = jnp.round(r[:, 4:5] * s)
    bsw = jnp.maximum(x2 - x1 + 1.0, 1.0) * (1.0 / POOLED)
    bsh = jnp.maximum(y2 - y1 + 1.0, 1.0) * (1.0 / POOLED)
    ws = jnp.clip(jnp.floor(pw * bsw) + x1, 0.0, hwf)
    we = jnp.clip(jnp.ceil((pw + 1.0) * bsw) + x1, 0.0, hwf)
    hs = jnp.clip(jnp.floor(ph * bsh) + y1, 0.0, hwf)
    he = jnp.clip(jnp.ceil((ph + 1.0) * bsh) + y1, 0.0, hwf)
    valid = (ws < we) & (hs < he)

    hwi = hwf.astype(jnp.int32)
    base = basef.astype(jnp.int32)
    wsi = ws.astype(jnp.int32)
    sw = we.astype(jnp.int32) - wsi
    h = jnp.minimum(hs.astype(jnp.int32) + di, he.astype(jnp.int32) - 1)
    idx = (sw - 1) * NT_PAD + base + (b * hwi + h) * hwi + wsi
    idx_ref[...] = jnp.where(valid, idx, ZROW)


_idx_call = pl.pallas_call(
    _idx_body,
    out_shape=jax.ShapeDtypeStruct((NROI, IDX_PER_ROI), jnp.int32),
)


def _span_body(cur_ref, nxt_ref, out_ref):
    blk = jnp.concatenate([cur_ref[...], nxt_ref[:8]], axis=0)
    m = blk[:RB]
    out_ref[0] = m
    for sft in (1, 2, 3):
        m = jnp.maximum(m, lax.slice_in_dim(blk, sft, sft + RB, axis=0))
        out_ref[sft] = m


_span_call = pl.pallas_call(
    _span_body,
    grid=(16,),
    in_specs=[
        pl.BlockSpec((RB, CH), lambda i: (i, 0)),
        pl.BlockSpec((RB, CH), lambda i: (i + 1, 0)),
    ],
    out_specs=pl.BlockSpec((NSPAN, RB, CH), lambda i: (0, i, 0)),
    out_shape=jax.ShapeDtypeStruct((NSPAN, NT_PAD, CH), jnp.float32),
)


def _sc_body(table_hbm, idx_hbm, out_hbm, idx_v,
             buf0, buf1, buf2, buf3, orow, sem0, sem1, sem2, sem3):
    cid = lax.axis_index("c")
    sid = lax.axis_index("s")
    wid = sid * 2 + cid
    bin0 = wid * BPT
    pltpu.sync_copy(idx_hbm.at[pl.ds(bin0 * 4, BPT * 4)], idx_v)

    bufs = (buf0, buf1, buf2, buf3)
    sems = (sem0, sem1, sem2, sem3)

    def gather(ch, buf, sem):
        return pltpu.make_async_copy(
            table_hbm.at[idx_v.at[pl.ds(ch * ROWS_PER_CHUNK, ROWS_PER_CHUNK)]],
            buf, sem)

    for b in range(NBUF):
        gather(b, bufs[b], sems[b]).start()

    def group(it, carry):
        ch0 = it * NBUF
        for b in range(NBUF):
            ch = ch0 + b
            buf, sem = bufs[b], sems[b]
            gather(ch, buf, sem).wait()
            for k in range(CB):
                for c in range(NVREG):
                    acc = buf[k * 4, pl.ds(c * 16, 16)]
                    for rr in range(1, 4):
                        acc = jnp.maximum(acc, buf[k * 4 + rr, pl.ds(c * 16, 16)])
                    orow[k, pl.ds(c * 16, 16)] = acc
            pltpu.sync_copy(orow, out_hbm.at[pl.ds(bin0 + ch * CB, CB)])
            gather(jnp.minimum(ch + NBUF, NCHUNKS - 1), buf, sem).start()
        return carry

    lax.fori_loop(0, NCHUNKS // NBUF, group, 0)
    for b in range(NBUF):
        gather(0, bufs[b], sems[b]).wait()


@functools.cache
def _sc_call():
    return pl.kernel(
        _sc_body,
        out_type=jax.ShapeDtypeStruct((NBINS_PAD, CH), jnp.float32),
        mesh=plsc.VectorSubcoreMesh(core_axis_name="c", subcore_axis_name="s"),
        scratch_types=(
            [pltpu.VMEM((BPT * 4,), jnp.int32)]
            + [pltpu.VMEM((ROWS_PER_CHUNK, CH), jnp.float32)] * NBUF
            + [pltpu.VMEM((CB, CH), jnp.float32)]
            + [pltpu.SemaphoreType.DMA] * NBUF
        ),
        compiler_params=pltpu.CompilerParams(use_tc_tiling_on_sc=False),
    )


def kernel(feat_p2, feat_p3, feat_p4, feat_p5, feat_p6,
           rois_p2, rois_p3, rois_p4, rois_p5, rois_p6):
    feats = (feat_p2, feat_p3, feat_p4, feat_p5, feat_p6)
    roiss = (rois_p2, rois_p3, rois_p4, rois_p5, rois_p6)

    rows = [jnp.transpose(f, (0, 2, 3, 1)).reshape(-1, CH) for f in feats]
    table = jnp.concatenate(
        rows + [jnp.zeros((NT_PAD + 8 - NROWS, CH), jnp.float32)], axis=0)
    span = _span_call(table, table).reshape(NSPAN * NT_PAD, CH)

    rois_all = jnp.concatenate(roiss, axis=0)
    par = jnp.concatenate([
        jnp.broadcast_to(
            jnp.array([1.0 / st, float(hw), float(bs), 0.0, 0.0, 0.0, 0.0, 0.0],
                      jnp.float32)[None, :], (NROI_L, 8))
        for st, hw, bs in zip(STRIDES, LEVEL_HW, _BASES)
    ], axis=0)

    idx = _idx_call(rois_all, par).reshape(-1)
    idx = jnp.concatenate(
        [idx, jnp.full((NBINS_PAD * 4 - NBINS * 4,), ZROW, jnp.int32)])

    pooled = _sc_call()(span, idx)
    out = pooled[:NBINS].reshape(NROI, POOLED, POOLED, CH)
    return jnp.transpose(out, (0, 3, 1, 2))


# speedup vs baseline: 1.0048x; 1.0048x over previous
import functools

import jax
import jax.numpy as jnp
from jax import lax
from jax.experimental import pallas as pl
from jax.experimental.pallas import tpu as pltpu
from jax.experimental.pallas import tpu_sc as plsc

POOLED = 7
STRIDES = (4, 8, 16, 32, 64)
LEVEL_HW = (96, 48, 24, 12, 6)
CH = 192
NROI_L = 64
NROI = NROI_L * 5
BINS_PER_ROI = POOLED * POOLED
IDX_PER_ROI = BINS_PER_ROI * 4
NBINS = NROI * BINS_PER_ROI
NTILES = 32
BPT = 512
NBINS_PAD = BPT * NTILES
CB = 16
ROWS_PER_CHUNK = CB * 4
NCHUNKS = BPT // CB
NBUF = 4
NVREG = CH // 32

_BASES = []
_off = 0
for _hw in LEVEL_HW:
    _BASES.append(_off)
    _off += 2 * _hw * _hw
NROWS = _off
ZROW = NROWS
RB = 1536
NT_PAD = RB * 16
NSPAN = 4


def _idx_body(rois_ref, par_ref, idx_ref):
    col = lax.broadcasted_iota(jnp.int32, (NROI, IDX_PER_ROI), 1)
    ph = (col // 28).astype(jnp.float32)
    pw = ((col % 28) // 4).astype(jnp.float32)
    di = col % 4

    r = rois_ref[...]
    p = par_ref[...]
    b = r[:, 0:1].astype(jnp.int32)
    s = p[:, 0:1]
    hwf = p[:, 1:2]
    basef = p[:, 2:3]
    x1 = jnp.round(r[:, 1:2] * s)
    y1 = jnp.round(r[:, 2:3] * s)
    x2 = jnp.round(r[:, 3:4] * s)
    y2 = jnp.round(r[:, 4:5] * s)
    bsw = jnp.maximum(x2 - x1 + 1.0, 1.0) * (1.0 / POOLED)
    bsh = jnp.maximum(y2 - y1 + 1.0, 1.0) * (1.0 / POOLED)
    ws = jnp.clip(jnp.floor(pw * bsw) + x1, 0.0, hwf)
    we = jnp.clip(jnp.ceil((pw + 1.0) * bsw) + x1, 0.0, hwf)
    hs = jnp.clip(jnp.floor(ph * bsh) + y1, 0.0, hwf)
    he = jnp.clip(jnp.ceil((ph + 1.0) * bsh) + y1, 0.0, hwf)
    valid = (ws < we) & (hs < he)

    hwi = hwf.astype(jnp.int32)
    base = basef.astype(jnp.int32)
    wsi = ws.astype(jnp.int32)
    sw = we.astype(jnp.int32) - wsi
    h = jnp.minimum(hs.astype(jnp.int32) + di, he.astype(jnp.int32) - 1)
    idx = (sw - 1) * NT_PAD + base + (b * hwi + h) * hwi + wsi
    idx_ref[...] = jnp.where(valid, idx, ZROW)


_idx_call = pl.pallas_call(
    _idx_body,
    out_shape=jax.ShapeDtypeStruct((NROI, IDX_PER_ROI), jnp.int32),
)


def _span_body(cur_ref, nxt_ref, out_ref):
    blk = jnp.concatenate([cur_ref[...], nxt_ref[:8]], axis=0)
    m = blk[:RB]
    out_ref[0] = m
    for sft in (1, 2, 3):
        m = jnp.maximum(m, lax.slice_in_dim(blk, sft, sft + RB, axis=0))
        out_ref[sft] = m


_span_call = pl.pallas_call(
    _span_body,
    grid=(16,),
    in_specs=[
        pl.BlockSpec((RB, CH), lambda i: (i, 0)),
        pl.BlockSpec((RB, CH), lambda i: (i + 1, 0)),
    ],
    out_specs=pl.BlockSpec((NSPAN, RB, CH), lambda i: (0, i, 0)),
    out_shape=jax.ShapeDtypeStruct((NSPAN, NT_PAD, CH), jnp.bfloat16),
)


def _sc_body(table_hbm, idx_hbm, out_hbm, idx_v,
             buf0, buf1, buf2, buf3, orow, sem0, sem1, sem2, sem3):
    cid = lax.axis_index("c")
    sid = lax.axis_index("s")
    wid = sid * 2 + cid
    bin0 = wid * BPT
    pltpu.sync_copy(idx_hbm.at[pl.ds(bin0 * 4, BPT * 4)], idx_v)

    bufs = (buf0, buf1, buf2, buf3)
    sems = (sem0, sem1, sem2, sem3)

    def gather(ch, buf, sem):
        return pltpu.make_async_copy(
            table_hbm.at[idx_v.at[pl.ds(ch * ROWS_PER_CHUNK, ROWS_PER_CHUNK)]],
            buf, sem)

    for b in range(NBUF):
        gather(b, bufs[b], sems[b]).start()

    def group(it, carry):
        ch0 = it * NBUF
        for b in range(NBUF):
            ch = ch0 + b
            buf, sem = bufs[b], sems[b]
            gather(ch, buf, sem).wait()
            for k in range(CB):
                for c in range(NVREG):
                    acc = buf[k * 4, pl.ds(c * 32, 32)]
                    for rr in range(1, 4):
                        acc = jnp.maximum(acc, buf[k * 4 + rr, pl.ds(c * 32, 32)])
                    orow[k, pl.ds(c * 32, 32)] = acc
            pltpu.sync_copy(orow, out_hbm.at[pl.ds(bin0 + ch * CB, CB)])
            gather(jnp.minimum(ch + NBUF, NCHUNKS - 1), buf, sem).start()
        return carry

    lax.fori_loop(0, NCHUNKS // NBUF, group, 0)
    for b in range(NBUF):
        gather(0, bufs[b], sems[b]).wait()


@functools.cache
def _sc_call():
    return pl.kernel(
        _sc_body,
        out_type=jax.ShapeDtypeStruct((NBINS_PAD, CH), jnp.bfloat16),
        mesh=plsc.VectorSubcoreMesh(core_axis_name="c", subcore_axis_name="s"),
        scratch_types=(
            [pltpu.VMEM((BPT * 4,), jnp.int32)]
            + [pltpu.VMEM((ROWS_PER_CHUNK, CH), jnp.bfloat16)] * NBUF
            + [pltpu.VMEM((CB, CH), jnp.bfloat16)]
            + [pltpu.SemaphoreType.DMA] * NBUF
        ),
        compiler_params=pltpu.CompilerParams(use_tc_tiling_on_sc=False),
    )


def kernel(feat_p2, feat_p3, feat_p4, feat_p5, feat_p6,
           rois_p2, rois_p3, rois_p4, rois_p5, rois_p6):
    feats = (feat_p2, feat_p3, feat_p4, feat_p5, feat_p6)
    roiss = (rois_p2, rois_p3, rois_p4, rois_p5, rois_p6)

    rows = [jnp.transpose(f.astype(jnp.bfloat16), (0, 2, 3, 1)).reshape(-1, CH)
            for f in feats]
    table = jnp.concatenate(
        rows + [jnp.zeros((NT_PAD + 8 - NROWS, CH), jnp.bfloat16)], axis=0)
    span = _span_call(table, table).reshape(NSPAN * NT_PAD, CH)

    rois_all = jnp.concatenate(roiss, axis=0)
    par = jnp.concatenate([
        jnp.broadcast_to(
            jnp.array([1.0 / st, float(hw), float(bs), 0.0, 0.0, 0.0, 0.0, 0.0],
                      jnp.float32)[None, :], (NROI_L, 8))
        for st, hw, bs in zip(STRIDES, LEVEL_HW, _BASES)
    ], axis=0)

    idx = _idx_call(rois_all, par).reshape(-1)
    idx = jnp.concatenate(
        [idx, jnp.full((NBINS_PAD * 4 - NBINS * 4,), ZROW, jnp.int32)])

    pooled = _sc_call()(span, idx)
    out = pooled[:NBINS].reshape(NROI, POOLED, POOLED, CH)
    return jnp.transpose(out, (0, 3, 1, 2)).astype(jnp.float32)


# speedup vs baseline: 1.0356x; 1.0306x over previous
import functools

import jax
import jax.numpy as jnp
from jax import lax
from jax.experimental import pallas as pl
from jax.experimental.pallas import tpu as pltpu
from jax.experimental.pallas import tpu_sc as plsc

POOLED = 7
STRIDES = (4, 8, 16, 32, 64)
LEVEL_HW = (96, 48, 24, 12, 6)
CH = 192
NROI_L = 64
NROI = NROI_L * 5
BINS_PER_ROI = POOLED * POOLED
IDX_PER_ROI = BINS_PER_ROI * 4
NBINS = NROI * BINS_PER_ROI
NTILES = 32
BPT = 512
NBINS_PAD = BPT * NTILES
CB = 16
ROWS_PER_CHUNK = CB * 4
NCHUNKS = BPT // CB
NBUF = 4
NVREG = CH // 32

_BASES = []
_off = 0
for _hw in LEVEL_HW:
    _BASES.append(_off)
    _off += 2 * _hw * _hw
NROWS = _off
ZROW = NROWS
RB = 1536
NT_PAD = RB * 16
NSPAN = 4


def _idx_body(rois_ref, par_ref, idx_ref):
    col = lax.broadcasted_iota(jnp.int32, (NROI, IDX_PER_ROI), 1)
    ph = (col // 28).astype(jnp.float32)
    pw = ((col % 28) // 4).astype(jnp.float32)
    di = col % 4

    r = rois_ref[...]
    p = par_ref[...]
    b = r[:, 0:1].astype(jnp.int32)
    s = p[:, 0:1]
    hwf = p[:, 1:2]
    basef = p[:, 2:3]
    x1 = jnp.round(r[:, 1:2] * s)
    y1 = jnp.round(r[:, 2:3] * s)
    x2 = jnp.round(r[:, 3:4] * s)
    y2 = jnp.round(r[:, 4:5] * s)
    bsw = jnp.maximum(x2 - x1 + 1.0, 1.0) * (1.0 / POOLED)
    bsh = jnp.maximum(y2 - y1 + 1.0, 1.0) * (1.0 / POOLED)
    ws = jnp.clip(jnp.floor(pw * bsw) + x1, 0.0, hwf)
    we = jnp.clip(jnp.ceil((pw + 1.0) * bsw) + x1, 0.0, hwf)
    hs = jnp.clip(jnp.floor(ph * bsh) + y1, 0.0, hwf)
    he = jnp.clip(jnp.ceil((ph + 1.0) * bsh) + y1, 0.0, hwf)
    valid = (ws < we) & (hs < he)

    hwi = hwf.astype(jnp.int32)
    base = basef.astype(jnp.int32)
    wsi = ws.astype(jnp.int32)
    sw = we.astype(jnp.int32) - wsi
    h = jnp.minimum(hs.astype(jnp.int32) + di, he.astype(jnp.int32) - 1)
    idx = (sw - 1) * NT_PAD + base + (b * hwi + h) * hwi + wsi
    idx_ref[...] = jnp.where(valid, idx, ZROW)


_idx_call = pl.pallas_call(
    _idx_body,
    out_shape=jax.ShapeDtypeStruct((NROI, IDX_PER_ROI), jnp.int32),
)


def _span_body(cur_ref, nxt_ref, out_ref):
    blk = jnp.concatenate([cur_ref[...], nxt_ref[:8]], axis=0)
    m = blk[:RB]
    out_ref[0] = m.astype(jnp.bfloat16)
    for sft in (1, 2, 3):
        m = jnp.maximum(m, lax.slice_in_dim(blk, sft, sft + RB, axis=0))
        out_ref[sft] = m.astype(jnp.bfloat16)


_span_call = pl.pallas_call(
    _span_body,
    grid=(16,),
    in_specs=[
        pl.BlockSpec((RB, CH), lambda i: (i, 0)),
        pl.BlockSpec((RB, CH), lambda i: (i + 1, 0)),
    ],
    out_specs=pl.BlockSpec((NSPAN, RB, CH), lambda i: (0, i, 0)),
    out_shape=jax.ShapeDtypeStruct((NSPAN, NT_PAD, CH), jnp.bfloat16),
)


BR = 8


def _outt_body(in_ref, out_ref):
    x = in_ref[...].reshape(BR, BINS_PER_ROI, CH)
    out_ref[...] = jnp.transpose(x, (0, 2, 1)).astype(jnp.float32)


_outt_call = pl.pallas_call(
    _outt_body,
    grid=(NROI // BR,),
    in_specs=[pl.BlockSpec((BINS_PER_ROI * BR, CH), lambda i: (i, 0))],
    out_specs=pl.BlockSpec((BR, CH, BINS_PER_ROI), lambda i: (i, 0, 0)),
    out_shape=jax.ShapeDtypeStruct((NROI, CH, BINS_PER_ROI), jnp.float32),
)


def _sc_body(table_hbm, idx_hbm, out_hbm, idx_v,
             buf0, buf1, buf2, buf3, orow, sem0, sem1, sem2, sem3):
    cid = lax.axis_index("c")
    sid = lax.axis_index("s")
    wid = sid * 2 + cid
    bin0 = wid * BPT
    pltpu.sync_copy(idx_hbm.at[pl.ds(bin0 * 4, BPT * 4)], idx_v)

    bufs = (buf0, buf1, buf2, buf3)
    sems = (sem0, sem1, sem2, sem3)

    def gather(ch, buf, sem):
        return pltpu.make_async_copy(
            table_hbm.at[idx_v.at[pl.ds(ch * ROWS_PER_CHUNK, ROWS_PER_CHUNK)]],
            buf, sem)

    for b in range(NBUF):
        gather(b, bufs[b], sems[b]).start()

    def group(it, carry):
        ch0 = it * NBUF
        for b in range(NBUF):
            ch = ch0 + b
            buf, sem = bufs[b], sems[b]
            gather(ch, buf, sem).wait()
            for k in range(CB):
                for c in range(NVREG):
                    acc = buf[k * 4, pl.ds(c * 32, 32)]
                    for rr in range(1, 4):
                        acc = jnp.maximum(acc, buf[k * 4 + rr, pl.ds(c * 32, 32)])
                    orow[k, pl.ds(c * 32, 32)] = acc
            pltpu.sync_copy(orow, out_hbm.at[pl.ds(bin0 + ch * CB, CB)])
            gather(jnp.minimum(ch + NBUF, NCHUNKS - 1), buf, sem).start()
        return carry

    lax.fori_loop(0, NCHUNKS // NBUF, group, 0)
    for b in range(NBUF):
        gather(0, bufs[b], sems[b]).wait()


@functools.cache
def _sc_call():
    return pl.kernel(
        _sc_body,
        out_type=jax.ShapeDtypeStruct((NBINS_PAD, CH), jnp.bfloat16),
        mesh=plsc.VectorSubcoreMesh(core_axis_name="c", subcore_axis_name="s"),
        scratch_types=(
            [pltpu.VMEM((BPT * 4,), jnp.int32)]
            + [pltpu.VMEM((ROWS_PER_CHUNK, CH), jnp.bfloat16)] * NBUF
            + [pltpu.VMEM((CB, CH), jnp.bfloat16)]
            + [pltpu.SemaphoreType.DMA] * NBUF
        ),
        compiler_params=pltpu.CompilerParams(use_tc_tiling_on_sc=False),
    )


def kernel(feat_p2, feat_p3, feat_p4, feat_p5, feat_p6,
           rois_p2, rois_p3, rois_p4, rois_p5, rois_p6):
    feats = (feat_p2, feat_p3, feat_p4, feat_p5, feat_p6)
    roiss = (rois_p2, rois_p3, rois_p4, rois_p5, rois_p6)

    rows = [jnp.transpose(f, (0, 2, 3, 1)).reshape(-1, CH) for f in feats]
    table = jnp.concatenate(
        rows + [jnp.zeros((NT_PAD + 8 - NROWS, CH), jnp.float32)], axis=0)
    span = _span_call(table, table).reshape(NSPAN * NT_PAD, CH)

    rois_all = jnp.concatenate(roiss, axis=0)
    par = jnp.concatenate([
        jnp.broadcast_to(
            jnp.array([1.0 / st, float(hw), float(bs), 0.0, 0.0, 0.0, 0.0, 0.0],
                      jnp.float32)[None, :], (NROI_L, 8))
        for st, hw, bs in zip(STRIDES, LEVEL_HW, _BASES)
    ], axis=0)

    idx = _idx_call(rois_all, par).reshape(-1)
    idx = jnp.concatenate(
        [idx, jnp.full((NBINS_PAD * 4 - NBINS * 4,), ZROW, jnp.int32)])

    pooled = _sc_call()(span, idx)
    return _outt_call(pooled).reshape(NROI, CH, POOLED, POOLED)


# speedup vs baseline: 1.0498x; 1.0137x over previous
import functools

import jax
import jax.numpy as jnp
from jax import lax
from jax.experimental import pallas as pl
from jax.experimental.pallas import tpu as pltpu
from jax.experimental.pallas import tpu_sc as plsc

POOLED = 7
STRIDES = (4, 8, 16, 32, 64)
LEVEL_HW = (96, 48, 24, 12, 6)
CH = 192
NROI_L = 64
NROI = NROI_L * 5
BINS_PER_ROI = POOLED * POOLED
IDX_PER_ROI = BINS_PER_ROI * 4
NBINS = NROI * BINS_PER_ROI
NTILES = 32
BPT = 512
NBINS_PAD = BPT * NTILES
CB = 16
ROWS_PER_CHUNK = CB * 4
NCHUNKS = BPT // CB
NBUF = 4
CW = CH // 2
NVREG = CW // 16

_BASES = []
_off = 0
for _hw in LEVEL_HW:
    _BASES.append(_off)
    _off += 2 * _hw * _hw
NROWS = _off
ZROW = NROWS
RB = 1536
NT_PAD = RB * 16
NSPAN = 4


def _idx_body(rois_ref, par_ref, idx_ref):
    col = lax.broadcasted_iota(jnp.int32, (NROI, IDX_PER_ROI), 1)
    ph = (col // 28).astype(jnp.float32)
    pw = ((col % 28) // 4).astype(jnp.float32)
    di = col % 4

    r = rois_ref[...]
    p = par_ref[...]
    b = r[:, 0:1].astype(jnp.int32)
    s = p[:, 0:1]
    hwf = p[:, 1:2]
    basef = p[:, 2:3]
    x1 = jnp.round(r[:, 1:2] * s)
    y1 = jnp.round(r[:, 2:3] * s)
    x2 = jnp.round(r[:, 3:4] * s)
    y2 = jnp.round(r[:, 4:5] * s)
    bsw = jnp.maximum(x2 - x1 + 1.0, 1.0) * (1.0 / POOLED)
    bsh = jnp.maximum(y2 - y1 + 1.0, 1.0) * (1.0 / POOLED)
    ws = jnp.clip(jnp.floor(pw * bsw) + x1, 0.0, hwf)
    we = jnp.clip(jnp.ceil((pw + 1.0) * bsw) + x1, 0.0, hwf)
    hs = jnp.clip(jnp.floor(ph * bsh) + y1, 0.0, hwf)
    he = jnp.clip(jnp.ceil((ph + 1.0) * bsh) + y1, 0.0, hwf)
    valid = (ws < we) & (hs < he)

    hwi = hwf.astype(jnp.int32)
    base = basef.astype(jnp.int32)
    wsi = ws.astype(jnp.int32)
    sw = we.astype(jnp.int32) - wsi
    h = jnp.minimum(hs.astype(jnp.int32) + di, he.astype(jnp.int32) - 1)
    idx = (sw - 1) * NT_PAD + base + (b * hwi + h) * hwi + wsi
    idx_ref[...] = jnp.where(valid, idx, ZROW)


_idx_call = pl.pallas_call(
    _idx_body,
    out_shape=jax.ShapeDtypeStruct((NROI, IDX_PER_ROI), jnp.int32),
)


def _span_body(cur_ref, nxt_ref, out_ref):
    blk = jnp.concatenate([cur_ref[...], nxt_ref[:8]], axis=0)
    m = blk[:RB]
    out_ref[0] = m
    for sft in (1, 2, 3):
        m = jnp.maximum(m, lax.slice_in_dim(blk, sft, sft + RB, axis=0))
        out_ref[sft] = m


_span_call = pl.pallas_call(
    _span_body,
    grid=(16,),
    in_specs=[
        pl.BlockSpec((RB, CH), lambda i: (i, 0)),
        pl.BlockSpec((RB, CH), lambda i: (i + 1, 0)),
    ],
    out_specs=pl.BlockSpec((NSPAN, RB, CH), lambda i: (0, i, 0)),
    out_shape=jax.ShapeDtypeStruct((NSPAN, NT_PAD, CH), jnp.float32),
)


BR = 8


def _outt_body(in_ref, out_ref):
    x = in_ref[...].reshape(BR, BINS_PER_ROI, CH)
    out_ref[...] = jnp.transpose(x, (0, 2, 1))


_outt_call = pl.pallas_call(
    _outt_body,
    grid=(NROI // BR,),
    in_specs=[pl.BlockSpec((BINS_PER_ROI * BR, CH), lambda i: (i, 0))],
    out_specs=pl.BlockSpec((BR, CH, BINS_PER_ROI), lambda i: (i, 0, 0)),
    out_shape=jax.ShapeDtypeStruct((NROI, CH, BINS_PER_ROI), jnp.float32),
)


def _sc_body(table_hbm, idx_hbm, out_hbm, idx_v,
             buf0, buf1, buf2, buf3, orow, sem0, sem1, sem2, sem3):
    cid = lax.axis_index("c")
    sid = lax.axis_index("s")
    wid = sid * 2 + cid
    bin0 = wid * BPT
    pltpu.sync_copy(idx_hbm.at[pl.ds(bin0 * 4, BPT * 4)], idx_v)

    bufs = (buf0, buf1, buf2, buf3)
    sems = (sem0, sem1, sem2, sem3)

    def gather(ch, buf, sem):
        return pltpu.make_async_copy(
            table_hbm.at[idx_v.at[pl.ds(ch * ROWS_PER_CHUNK, ROWS_PER_CHUNK)]],
            buf, sem)

    for b in range(NBUF):
        gather(b, bufs[b], sems[b]).start()

    def group(it, carry):
        ch0 = it * NBUF
        for b in range(NBUF):
            ch = ch0 + b
            buf, sem = bufs[b], sems[b]
            gather(ch, buf, sem).wait()
            for k in range(CB):
                for c in range(NVREG):
                    acc = buf[k * 4, pl.ds(c * 16, 16)]
                    for rr in range(1, 4):
                        acc = jnp.maximum(acc, buf[k * 4 + rr, pl.ds(c * 16, 16)])
                    orow[k, pl.ds(c * 16, 16)] = acc
            pltpu.sync_copy(orow, out_hbm.at[pl.ds(bin0 + ch * CB, CB)])
            gather(jnp.minimum(ch + NBUF, NCHUNKS - 1), buf, sem).start()
        return carry

    lax.fori_loop(0, NCHUNKS // NBUF, group, 0)
    for b in range(NBUF):
        gather(0, bufs[b], sems[b]).wait()


@functools.cache
def _sc_call():
    return pl.kernel(
        _sc_body,
        out_type=jax.ShapeDtypeStruct((NBINS_PAD, CH), jnp.float32),
        mesh=plsc.VectorSubcoreMesh(core_axis_name="c", subcore_axis_name="s"),
        scratch_types=(
            [pltpu.VMEM((BPT * 4,), jnp.int32)]
            + [pltpu.VMEM((ROWS_PER_CHUNK, CH), jnp.float32)] * NBUF
            + [pltpu.VMEM((CB, CH), jnp.float32)]
            + [pltpu.SemaphoreType.DMA] * NBUF
        ),
        compiler_params=pltpu.CompilerParams(use_tc_tiling_on_sc=False),
    )


def kernel(feat_p2, feat_p3, feat_p4, feat_p5, feat_p6,
           rois_p2, rois_p3, rois_p4, rois_p5, rois_p6):
    feats = (feat_p2, feat_p3, feat_p4, feat_p5, feat_p6)
    roiss = (rois_p2, rois_p3, rois_p4, rois_p5, rois_p6)

    rows = [jnp.transpose(f, (0, 2, 3, 1)).reshape(-1, CH) for f in feats]
    table = jnp.concatenate(
        rows + [jnp.zeros((NT_PAD + 8 - NROWS, CH), jnp.float32)], axis=0)
    span = _span_call(table, table).reshape(NSPAN * NT_PAD, CH)

    rois_all = jnp.concatenate(roiss, axis=0)
    par = jnp.concatenate([
        jnp.broadcast_to(
            jnp.array([1.0 / st, float(hw), float(bs), 0.0, 0.0, 0.0, 0.0, 0.0],
                      jnp.float32)[None, :], (NROI_L, 8))
        for st, hw, bs in zip(STRIDES, LEVEL_HW, _BASES)
    ], axis=0)

    idx = _idx_call(rois_all, par).reshape(-1)
    idx = jnp.concatenate(
        [idx, jnp.full((NBINS_PAD * 4 - NBINS * 4,), ZROW, jnp.int32)])

    pooled = _sc_call()(span, idx)
    return _outt_call(pooled).reshape(NROI, CH, POOLED, POOLED)
